# speedup vs baseline: 140.3512x; 140.3512x over previous
import functools

import jax
import jax.numpy as jnp
from jax import lax
from jax.experimental import pallas as pl
from jax.experimental.pallas import tpu as pltpu
from jax.experimental.pallas import tpu_sc as plsc

_VOCAB = 1000000
_B = 16384
_L = 200
_TOTAL = _B * _L
_NC = 2
_NS = 16
_NW = _NC * _NS
_PER_TILE = _TOTAL // _NW
_C = 4096
_NCHUNK = _PER_TILE // _C
_LANES = 16


def _sc_body(arr_hbm, idx_hbm, out_hbm, x_v, i1_v, i2_v, a1_v, a2_v, o_v,
             sem1, sem2):
    cid = lax.axis_index("c")
    sid = lax.axis_index("s")
    wid = sid * _NC + cid
    base = wid * _PER_TILE

    def chunk_body(ci, carry):
        off = base + ci * _C
        pltpu.sync_copy(idx_hbm.at[pl.ds(off, _C)], x_v)

        def vec_body(j, c):
            sl = pl.ds(j * _LANES, _LANES)
            x = x_v[sl]
            i1 = x.astype(jnp.int32)
            frac = x - i1.astype(jnp.float32)
            i2 = jnp.where(frac != 0.0, i1 + 1, i1)
            i1_v[sl] = i1
            i2_v[sl] = i2
            x_v[sl] = frac
            return c

        lax.fori_loop(0, _C // _LANES, vec_body, 0, unroll=4)

        cp1 = pltpu.async_copy(arr_hbm.at[i1_v], a1_v, sem1)
        cp2 = pltpu.async_copy(arr_hbm.at[i2_v], a2_v, sem2)
        cp1.wait()
        cp2.wait()

        def lerp_body(j, c):
            sl = pl.ds(j * _LANES, _LANES)
            t = x_v[sl]
            o_v[sl] = t * a1_v[sl] + (1.0 - t) * a2_v[sl]
            return c

        lax.fori_loop(0, _C // _LANES, lerp_body, 0, unroll=4)

        pltpu.sync_copy(o_v, out_hbm.at[pl.ds(off, _C)])
        return carry

    lax.fori_loop(0, _NCHUNK, chunk_body, 0)


@jax.jit
def kernel(arr, index):
    idx_flat = index.reshape(-1)
    mesh = plsc.VectorSubcoreMesh(core_axis_name="c", subcore_axis_name="s")
    out = pl.kernel(
        _sc_body,
        mesh=mesh,
        out_type=jax.ShapeDtypeStruct((_TOTAL,), jnp.float32),
        scratch_types=[
            pltpu.VMEM((_C,), jnp.float32),
            pltpu.VMEM((_C,), jnp.int32),
            pltpu.VMEM((_C,), jnp.int32),
            pltpu.VMEM((_C,), jnp.float32),
            pltpu.VMEM((_C,), jnp.float32),
            pltpu.VMEM((_C,), jnp.float32),
            pltpu.SemaphoreType.DMA,
            pltpu.SemaphoreType.DMA,
        ],
    )(arr, idx_flat)
    return out.reshape(_B, _L)


# speedup vs baseline: 155.9016x; 1.1108x over previous
import jax
import jax.numpy as jnp
from jax import lax
from jax.experimental import pallas as pl
from jax.experimental.pallas import tpu as pltpu
from jax.experimental.pallas import tpu_sc as plsc

_VOCAB = 1000000
_B = 16384
_L = 200
_TOTAL = _B * _L
_NC = 2
_NS = 16
_NW = _NC * _NS
_PER_TILE = _TOTAL // _NW
_C = 6400
_NCHUNK = _PER_TILE // _C
_LANES = 16


def _sc_body(arr_hbm, idx_hbm, out_hbm,
             x0, i10, i20, a10, a20, o0,
             x1, i11, i21, a11, a21, o1,
             sg10, sg20, sg11, sg21):
    cid = lax.axis_index("c")
    sid = lax.axis_index("s")
    wid = sid * _NC + cid
    base = wid * _PER_TILE

    bufs = ((x0, i10, i20, a10, a20, o0, sg10, sg20),
            (x1, i11, i21, a11, a21, o1, sg11, sg21))

    def stage_a(buf, ci):
        x_v, i1_v, i2_v, a1_v, a2_v, _, s1, s2 = buf
        off = base + ci * _C
        pltpu.sync_copy(idx_hbm.at[pl.ds(off, _C)], x_v)

        def vec_body(j, c):
            sl = pl.ds(j * _LANES, _LANES)
            x = x_v[sl]
            i1 = x.astype(jnp.int32)
            frac = x - i1.astype(jnp.float32)
            i2 = jnp.where(frac != 0.0, i1 + 1, i1)
            i1_v[sl] = i1
            i2_v[sl] = i2
            x_v[sl] = frac
            return c

        lax.fori_loop(0, _C // _LANES, vec_body, 0, unroll=4)
        cp1 = pltpu.async_copy(arr_hbm.at[i1_v], a1_v, s1)
        cp2 = pltpu.async_copy(arr_hbm.at[i2_v], a2_v, s2)
        return cp1, cp2

    def stage_b(buf, ci, cp1, cp2):
        x_v, _, _, a1_v, a2_v, o_v, _, _ = buf
        off = base + ci * _C
        cp1.wait()
        cp2.wait()

        def lerp_body(j, c):
            sl = pl.ds(j * _LANES, _LANES)
            t = x_v[sl]
            o_v[sl] = t * a1_v[sl] + (1.0 - t) * a2_v[sl]
            return c

        lax.fori_loop(0, _C // _LANES, lerp_body, 0, unroll=4)
        pltpu.sync_copy(o_v, out_hbm.at[pl.ds(off, _C)])

    def pair_body(it, carry):
        k0 = it * 2
        k1 = k0 + 1
        cps0 = stage_a(bufs[0], k0)
        cps1 = stage_a(bufs[1], k1)
        stage_b(bufs[0], k0, *cps0)
        stage_b(bufs[1], k1, *cps1)
        return carry

    lax.fori_loop(0, _NCHUNK // 2, pair_body, 0)


@jax.jit
def kernel(arr, index):
    idx_flat = index.reshape(-1)
    mesh = plsc.VectorSubcoreMesh(core_axis_name="c", subcore_axis_name="s")
    vmem_bufs = []
    for _ in range(2):
        vmem_bufs += [
            pltpu.VMEM((_C,), jnp.float32),
            pltpu.VMEM((_C,), jnp.int32),
            pltpu.VMEM((_C,), jnp.int32),
            pltpu.VMEM((_C,), jnp.float32),
            pltpu.VMEM((_C,), jnp.float32),
            pltpu.VMEM((_C,), jnp.float32),
        ]
    out = pl.kernel(
        _sc_body,
        mesh=mesh,
        out_type=jax.ShapeDtypeStruct((_TOTAL,), jnp.float32),
        scratch_types=vmem_bufs + [pltpu.SemaphoreType.DMA] * 4,
    )(arr, idx_flat)
    return out.reshape(_B, _L)


# speedup vs baseline: 244.4942x; 1.5683x over previous
import jax
import jax.numpy as jnp
from jax import lax
from jax.experimental import pallas as pl
from jax.experimental.pallas import tpu as pltpu
from jax.experimental.pallas import tpu_sc as plsc

_VOCAB = 1000000
_B = 16384
_L = 200
_TOTAL = _B * _L
_NC = 2
_NS = 16
_NW = _NC * _NS
_PER_TILE = _TOTAL // _NW
_C = 5120
_NCHUNK = _PER_TILE // _C
_LANES = 16


_STAGE = 62496
_STAGE_TAIL = _VOCAB - _NS * _STAGE


def _sc_body(arr_hbm, idx_hbm, out_hbm, tab_sh,
             x0, i10, i20, a10, a20, o0,
             x1, i11, i21, a11, a21, o1,
             sg10, sg20, sg11, sg21):
    cid = lax.axis_index("c")
    sid = lax.axis_index("s")
    wid = sid * _NC + cid
    base = wid * _PER_TILE

    soff = sid * _STAGE
    done = 0
    while done < _STAGE:
        sz = min(_C, _STAGE - done)
        pltpu.sync_copy(arr_hbm.at[pl.ds(soff + done, sz)], x0.at[pl.ds(0, sz)])
        pltpu.sync_copy(x0.at[pl.ds(0, sz)], tab_sh.at[pl.ds(soff + done, sz)])
        done += sz

    @pl.when(sid == _NS - 1)
    def _():
        tail = _NS * _STAGE
        pltpu.sync_copy(arr_hbm.at[pl.ds(tail, _STAGE_TAIL)], x1.at[pl.ds(0, _STAGE_TAIL)])
        pltpu.sync_copy(x1.at[pl.ds(0, _STAGE_TAIL)], tab_sh.at[pl.ds(tail, _STAGE_TAIL)])

    plsc.subcore_barrier()

    bufs = ((x0, i10, i20, a10, a20, o0, sg10, sg20),
            (x1, i11, i21, a11, a21, o1, sg11, sg21))

    def stage_a(buf, ci):
        x_v, i1_v, i2_v, a1_v, a2_v, _, s1, s2 = buf
        off = base + ci * _C
        pltpu.sync_copy(idx_hbm.at[pl.ds(off, _C)], x_v)

        def vec_body(j, c):
            sl = pl.ds(j * _LANES, _LANES)
            x = x_v[sl]
            i1 = x.astype(jnp.int32)
            frac = x - i1.astype(jnp.float32)
            i2 = jnp.where(frac != 0.0, i1 + 1, i1)
            i1_v[sl] = i1
            i2_v[sl] = i2
            x_v[sl] = frac
            return c

        lax.fori_loop(0, _C // _LANES, vec_body, 0, unroll=4)
        cp1 = pltpu.async_copy(tab_sh.at[i1_v], a1_v, s1)
        cp2 = pltpu.async_copy(tab_sh.at[i2_v], a2_v, s2)
        return cp1, cp2

    def stage_b(buf, ci, cp1, cp2):
        x_v, _, _, a1_v, a2_v, o_v, _, _ = buf
        off = base + ci * _C
        cp1.wait()
        cp2.wait()

        def lerp_body(j, c):
            sl = pl.ds(j * _LANES, _LANES)
            t = x_v[sl]
            o_v[sl] = t * a1_v[sl] + (1.0 - t) * a2_v[sl]
            return c

        lax.fori_loop(0, _C // _LANES, lerp_body, 0, unroll=4)
        pltpu.sync_copy(o_v, out_hbm.at[pl.ds(off, _C)])

    def pair_body(it, carry):
        k0 = it * 2
        k1 = k0 + 1
        cps0 = stage_a(bufs[0], k0)
        cps1 = stage_a(bufs[1], k1)
        stage_b(bufs[0], k0, *cps0)
        stage_b(bufs[1], k1, *cps1)
        return carry

    lax.fori_loop(0, _NCHUNK // 2, pair_body, 0)


@jax.jit
def kernel(arr, index):
    idx_flat = index.reshape(-1)
    mesh = plsc.VectorSubcoreMesh(core_axis_name="c", subcore_axis_name="s")
    vmem_bufs = []
    for _ in range(2):
        vmem_bufs += [
            pltpu.VMEM((_C,), jnp.float32),
            pltpu.VMEM((_C,), jnp.int32),
            pltpu.VMEM((_C,), jnp.int32),
            pltpu.VMEM((_C,), jnp.float32),
            pltpu.VMEM((_C,), jnp.float32),
            pltpu.VMEM((_C,), jnp.float32),
        ]
    out = pl.kernel(
        _sc_body,
        mesh=mesh,
        out_type=jax.ShapeDtypeStruct((_TOTAL,), jnp.float32),
        scratch_types=[pltpu.VMEM_SHARED((_VOCAB,), jnp.float32)]
        + vmem_bufs + [pltpu.SemaphoreType.DMA] * 4,
    )(arr, idx_flat)
    return out.reshape(_B, _L)
